# baseline (speedup 1.0000x reference)
import jax
import jax.numpy as jnp
from jax.experimental import pallas as pl
from jax.experimental.pallas import tpu as pltpu

B, C, H, W = 16, 13, 256, 256
MAXF = 64
IB = 4
BN = H * W
NB = B // IB


def _seg_kernel(lg_ref, fid_ref, lbl_ref, sums_ref, hist_ref):
    for i in range(IB):
        lg = lg_ref[i].astype(jnp.bfloat16).reshape(C, BN)
        fid = fid_ref[i].reshape(1, BN)
        lbl = lbl_ref[i].reshape(1, BN)

        oh_f = (jax.lax.broadcasted_iota(jnp.int32, (MAXF, BN), 0) == fid
                ).astype(jnp.bfloat16)
        oh_l = (jax.lax.broadcasted_iota(jnp.int32, (C, BN), 0) == lbl
                ).astype(jnp.bfloat16)

        dn = (((1,), (1,)), ((), ()))
        rhs = jnp.concatenate([lg, oh_l], axis=0)
        both = jax.lax.dot_general(oh_f, rhs, dn,
                                   preferred_element_type=jnp.float32)
        sums_ref[i] = both[:, :C]
        hist_ref[i] = both[:, C:]


def _ce_kernel(s_ref, h_ref, out_ref):
    s = s_ref[...]
    h = h_ref[...]
    R = B * MAXF
    counts = jnp.sum(h, axis=1, keepdims=True)
    mean = s / jnp.maximum(counts, 1.0)
    col = jax.lax.broadcasted_iota(jnp.int32, (R, C), 1)
    vh = jnp.where(col == 0, 0.0, h)
    has_valid = jnp.sum(vh, axis=1, keepdims=True) > 0.0
    hh = jnp.where(has_valid, vh, h)
    m = jnp.max(hh, axis=1, keepdims=True)
    label = jnp.min(jnp.where(hh == m, col, C), axis=1, keepdims=True)
    mx = jnp.max(mean, axis=1, keepdims=True)
    lse = jnp.log(jnp.sum(jnp.exp(mean - mx), axis=1, keepdims=True)) + mx
    sel = jnp.sum(jnp.where(col == label, mean, 0.0), axis=1, keepdims=True)
    ce = lse - sel
    fidx = jax.lax.broadcasted_iota(jnp.int32, (R, 1), 0)
    valid = ((counts > 0.0) & ((fidx & (MAXF - 1)) != 0)).astype(jnp.float32)
    t = jnp.sum(ce * valid).reshape(1, 1)
    n = jnp.sum(valid).reshape(1, 1)
    sa = jnp.sum(s).reshape(1, 1)
    out_ref[...] = jnp.where(n > 0.0, t / jnp.maximum(n, 1.0), sa * 0.0)


def kernel(logits, masks, field_ids):
    sums, hist = pl.pallas_call(
        _seg_kernel,
        grid=(NB,),
        in_specs=[
            pl.BlockSpec((IB, C, H, W), lambda j: (j, 0, 0, 0)),
            pl.BlockSpec((IB, H, W), lambda j: (j, 0, 0)),
            pl.BlockSpec((IB, H, W), lambda j: (j, 0, 0)),
        ],
        out_specs=[
            pl.BlockSpec((IB, MAXF, C), lambda j: (j, 0, 0)),
            pl.BlockSpec((IB, MAXF, C), lambda j: (j, 0, 0)),
        ],
        out_shape=[
            jax.ShapeDtypeStruct((B, MAXF, C), jnp.float32),
            jax.ShapeDtypeStruct((B, MAXF, C), jnp.float32),
        ],
    )(logits, field_ids, masks)

    out = pl.pallas_call(
        _ce_kernel,
        out_shape=jax.ShapeDtypeStruct((1, 1), jnp.float32),
    )(sums.reshape(B * MAXF, C), hist.reshape(B * MAXF, C))
    return out[0, 0]


# speedup vs baseline: 1.0300x; 1.0300x over previous
import jax
import jax.numpy as jnp
from jax.experimental import pallas as pl
from jax.experimental.pallas import tpu as pltpu

B, C, H, W = 16, 13, 256, 256
MAXF = 64
IB = 2
BN = H * W
NB = B // IB


def _seg_kernel(lg_ref, fid_ref, lbl_ref, sums_ref, hist_ref):
    for i in range(IB):
        lg = lg_ref[i].astype(jnp.bfloat16).reshape(C, BN)
        fid = fid_ref[i].reshape(1, BN)
        lbl = lbl_ref[i].reshape(1, BN)

        oh_f = (jax.lax.broadcasted_iota(jnp.int32, (MAXF, BN), 0) == fid
                ).astype(jnp.bfloat16)
        oh_l = (jax.lax.broadcasted_iota(jnp.int32, (C, BN), 0) == lbl
                ).astype(jnp.bfloat16)

        dn = (((1,), (1,)), ((), ()))
        rhs = jnp.concatenate([lg, oh_l], axis=0)
        both = jax.lax.dot_general(oh_f, rhs, dn,
                                   preferred_element_type=jnp.float32)
        sums_ref[i] = both[:, :C]
        hist_ref[i] = both[:, C:]


def _ce_kernel(s_ref, h_ref, out_ref):
    s = s_ref[...]
    h = h_ref[...]
    R = B * MAXF
    counts = jnp.sum(h, axis=1, keepdims=True)
    mean = s / jnp.maximum(counts, 1.0)
    col = jax.lax.broadcasted_iota(jnp.int32, (R, C), 1)
    vh = jnp.where(col == 0, 0.0, h)
    has_valid = jnp.sum(vh, axis=1, keepdims=True) > 0.0
    hh = jnp.where(has_valid, vh, h)
    m = jnp.max(hh, axis=1, keepdims=True)
    label = jnp.min(jnp.where(hh == m, col, C), axis=1, keepdims=True)
    mx = jnp.max(mean, axis=1, keepdims=True)
    lse = jnp.log(jnp.sum(jnp.exp(mean - mx), axis=1, keepdims=True)) + mx
    sel = jnp.sum(jnp.where(col == label, mean, 0.0), axis=1, keepdims=True)
    ce = lse - sel
    fidx = jax.lax.broadcasted_iota(jnp.int32, (R, 1), 0)
    valid = ((counts > 0.0) & ((fidx & (MAXF - 1)) != 0)).astype(jnp.float32)
    t = jnp.sum(ce * valid).reshape(1, 1)
    n = jnp.sum(valid).reshape(1, 1)
    sa = jnp.sum(s).reshape(1, 1)
    out_ref[...] = jnp.where(n > 0.0, t / jnp.maximum(n, 1.0), sa * 0.0)


def kernel(logits, masks, field_ids):
    sums, hist = pl.pallas_call(
        _seg_kernel,
        grid=(NB,),
        in_specs=[
            pl.BlockSpec((IB, C, H, W), lambda j: (j, 0, 0, 0)),
            pl.BlockSpec((IB, H, W), lambda j: (j, 0, 0)),
            pl.BlockSpec((IB, H, W), lambda j: (j, 0, 0)),
        ],
        out_specs=[
            pl.BlockSpec((IB, MAXF, C), lambda j: (j, 0, 0)),
            pl.BlockSpec((IB, MAXF, C), lambda j: (j, 0, 0)),
        ],
        out_shape=[
            jax.ShapeDtypeStruct((B, MAXF, C), jnp.float32),
            jax.ShapeDtypeStruct((B, MAXF, C), jnp.float32),
        ],
    )(logits, field_ids, masks)

    out = pl.pallas_call(
        _ce_kernel,
        out_shape=jax.ShapeDtypeStruct((1, 1), jnp.float32),
    )(sums.reshape(B * MAXF, C), hist.reshape(B * MAXF, C))
    return out[0, 0]
